# baseline (speedup 1.0000x reference)
import functools

import jax
import jax.numpy as jnp
from jax import lax
from jax.experimental import pallas as pl
from jax.experimental.pallas import tpu as pltpu
from jax.experimental.pallas import tpu_sc as plsc

Q = 1024
D = 16
K = 100000
KB = 1152
CHUNK_BLKS = 29
EPS = 1e-8


def _argmax_body(qt_ref, dbx_ref, dn_ref, qn_ref, out_ref, scores_sc, bv_sc, bi_sc,
                 av_sc, ai_sc):
    pid = pl.program_id(0)
    nblk = pl.num_programs(0)

    x = dbx_ref[...]
    xn = x * (1.0 / dn_ref[...])
    qt = qt_ref[...]
    qn = qt * (1.0 / qn_ref[...])
    s = lax.dot_general(xn, qn, (((1,), (0,)), ((), ())),
                        preferred_element_type=jnp.float32)
    scores_sc[...] = s

    @pl.when(pid == 0)
    def _init():
        bv_sc[...] = jnp.full((8, Q), -jnp.inf, jnp.float32)
        bi_sc[...] = jnp.zeros((8, Q), jnp.int32)
        av_sc[...] = jnp.full((1, Q), -jnp.inf, jnp.float32)
        ai_sc[...] = jnp.zeros((1, Q), jnp.int32)

    base = pid * KB
    iota8 = lax.broadcasted_iota(jnp.int32, (8, 1), 0)

    def slab(i, carry):
        bv, bi = carry
        off = pl.multiple_of(i * 8, 8)
        v = scores_sc[pl.ds(off, 8), :]
        rid = base + i * 8 + iota8
        v = v + jnp.where(rid < K, 0.0, -jnp.inf)
        upd = v > bv
        bv = jnp.where(upd, v, bv)
        bi = jnp.where(upd, rid, bi)
        return bv, bi

    bv, bi = lax.fori_loop(0, KB // 8, slab, (bv_sc[...], bi_sc[...]))
    bv_sc[...] = bv
    bi_sc[...] = bi

    @pl.when(jnp.logical_or(pid % CHUNK_BLKS == CHUNK_BLKS - 1, pid == nblk - 1))
    def _chunk_fold():
        mv = jnp.max(bv, axis=0, keepdims=True)
        cand = jnp.where(bv == jnp.broadcast_to(mv, (8, Q)), bi, K)
        ci = jnp.min(cand, axis=0, keepdims=True)
        av = av_sc[...]
        ai = ai_sc[...]
        upd = jnp.logical_or(mv > av, jnp.logical_and(mv == av, ci < ai))
        mq = mv.astype(jnp.bfloat16).astype(jnp.float32)
        av_sc[...] = jnp.where(upd, mq, av)
        ai_sc[...] = jnp.where(upd, ci, ai)
        bv_sc[...] = jnp.full((8, Q), -jnp.inf, jnp.float32)
        bi_sc[...] = jnp.zeros((8, Q), jnp.int32)

    @pl.when(pid == nblk - 1)
    def _finish():
        out_ref[...] = ai_sc[...]


def _argmax_indices(imu, database_x, dnorm, qnorm):
    nblk = pl.cdiv(K, KB)
    out = pl.pallas_call(
        _argmax_body,
        grid=(nblk,),
        in_specs=[
            pl.BlockSpec((D, Q), lambda i: (0, 0)),
            pl.BlockSpec((KB, D), lambda i: (i, 0)),
            pl.BlockSpec((KB, 1), lambda i: (i, 0)),
            pl.BlockSpec((1, Q), lambda i: (0, 0)),
        ],
        out_specs=pl.BlockSpec((1, Q), lambda i: (0, 0)),
        out_shape=jax.ShapeDtypeStruct((1, Q), jnp.int32),
        scratch_shapes=[
            pltpu.VMEM((KB, Q), jnp.float32),
            pltpu.VMEM((8, Q), jnp.float32),
            pltpu.VMEM((8, Q), jnp.int32),
            pltpu.VMEM((1, Q), jnp.float32),
            pltpu.VMEM((1, Q), jnp.int32),
        ],
    )(imu.T, database_x, dnorm, qnorm)
    return out.reshape(Q)


WROWS = 128 // D


@functools.lru_cache(maxsize=None)
def _make_gather():
    info = plsc.get_sparse_core_info()
    nc, ns = info.num_cores, info.num_subcores
    nw = nc * ns
    bpw = Q // nw
    mesh = plsc.VectorSubcoreMesh(core_axis_name="c", subcore_axis_name="s")

    @functools.partial(
        pl.kernel,
        mesh=mesh,
        out_type=jax.ShapeDtypeStruct((Q, D), jnp.float32),
        scratch_types=[
            pltpu.VMEM((bpw,), jnp.int32),
            pltpu.VMEM((bpw,), jnp.int32),
            pltpu.VMEM((16,), jnp.int32),
            pltpu.VMEM((bpw, 128), jnp.float32),
            pltpu.VMEM((bpw, D), jnp.float32),
            pltpu.SemaphoreType.DMA,
        ],
    )
    def gather_k(dby_hbm, idx_hbm, out_hbm, idx_v, g_v, tmp_v, wide_v, rows_v, sem):
        wid = lax.axis_index("s") * nc + lax.axis_index("c")
        base = wid * bpw
        pltpu.sync_copy(idx_hbm.at[pl.ds(base, bpw)], idx_v)
        for h in range(bpw // 16):
            iv = idx_v[pl.ds(h * 16, 16)]
            g_v[pl.ds(h * 16, 16)] = lax.shift_right_logical(iv, 3)
        pltpu.async_copy(dby_hbm.at[g_v], wide_v, sem).wait()
        for h in range(bpw // 16):
            iv = idx_v[pl.ds(h * 16, 16)]
            g_v[pl.ds(h * 16, 16)] = iv & 7
        for h in range(bpw // 16):
            for jl in range(16):
                j = h * 16 + jl
                ovb_raw = lax.gather(
                    g_v[pl.ds(h * 16, 16)], jnp.full((16, 1), jl, jnp.int32),
                    lax.GatherDimensionNumbers(
                        offset_dims=(), collapsed_slice_dims=(0,),
                        start_index_map=(0,)),
                    (1,), mode=lax.GatherScatterMode.PROMISE_IN_BOUNDS)
                tmp_v[...] = ovb_raw
                ovb = tmp_v[...]
                acc = wide_v[j, pl.ds(0, 16)]
                for s in range(1, WROWS):
                    seg = wide_v[j, pl.ds(s * D, 16)]
                    acc = jnp.where(ovb == s, seg, acc)
                rows_v[j, :] = acc
        pltpu.sync_copy(rows_v, out_hbm.at[pl.ds(base, bpw)])

    return gather_k


def kernel(imu, database_x, database_y):
    dnorm = jnp.maximum(jnp.linalg.norm(database_x, axis=-1, keepdims=True), EPS)
    qnorm = jnp.maximum(jnp.linalg.norm(imu, axis=-1, keepdims=True), EPS).reshape(1, Q)
    idx = _argmax_indices(imu, database_x, dnorm, qnorm)
    dby_wide = database_y.reshape(K // WROWS, 128)
    return _make_gather()(dby_wide, idx)
